# Initial kernel scaffold; baseline (speedup 1.0000x reference)
import jax
import jax.numpy as jnp
from jax.experimental import pallas as pl


def kernel(inp, visible_units, visible_fs, biases, L):
    raise NotImplementedError("write your pallas kernel here")


# speedup vs baseline: 10.8160x; 10.8160x over previous
import functools

import jax
import jax.numpy as jnp
from jax.experimental import pallas as pl
from jax.experimental.pallas import tpu as pltpu


def _scn_kernel(depth, TB, V, inp_ref, vu_ref, vf_ref, L_ref, b_ref, *out_refs):
    out_ref = out_refs[0]
    h_refs = out_refs[1::2]
    n_refs = out_refs[2::2]

    w_p = inp_ref[...]
    s = jnp.sum(w_p, axis=1, keepdims=True)
    iota = jax.lax.broadcasted_iota(jnp.int32, (TB, V), 1)
    w = jnp.where(iota == 0, 1.0 - s, w_p)
    f = jnp.broadcast_to(vf_ref[...], (TB, V))
    h = jnp.broadcast_to(vu_ref[...], (TB, V))

    for i in range(depth):
        l = L_ref[i : i + 1, :]
        wd = w / (l + 1e-20)
        m = jnp.min(wd, axis=1, keepdims=True)
        cand = jnp.where(wd == m, iota, V)
        idx = jnp.min(cand, axis=1, keepdims=True)
        mask = iota == idx
        lb = l.astype(jnp.bfloat16).astype(jnp.float32)
        hb = h.astype(jnp.bfloat16).astype(jnp.float32)
        fb = f.astype(jnp.bfloat16).astype(jnp.float32)
        s_h = jnp.sum(lb * hb, axis=1, keepdims=True)
        s_f = jnp.sum(lb * fb, axis=1, keepdims=True) + b_ref[i, 0]
        h_refs[i][...] = h
        n_refs[i][...] = s_h
        w = jnp.where(mask, m, w - m * l)
        h = jnp.where(mask, s_h, h)
        f = jnp.where(mask, s_f, f)

    out_ref[...] = jnp.sum(w * f, axis=1, keepdims=True)


@functools.partial(jax.jit, static_argnames=("TB",))
def _scn(inp, visible_units, visible_fs, biases, L, TB):
    B = inp.shape[0]
    V = visible_units.shape[0]
    depth = L.shape[0]
    inp_p = jnp.pad(inp, ((0, 0), (1, 0)))
    vu = visible_units.reshape(1, V)
    vf = visible_fs.reshape(1, V)

    grid = (B // TB,)
    row_spec = pl.BlockSpec((TB, V), lambda b: (b, 0))
    col_spec = pl.BlockSpec((TB, 1), lambda b: (b, 0))
    fix_spec = lambda shape: pl.BlockSpec(shape, lambda b: (0, 0))

    out_shapes = [jax.ShapeDtypeStruct((B, 1), jnp.float32)]
    out_specs = [col_spec]
    for _ in range(depth):
        out_shapes.append(jax.ShapeDtypeStruct((B, V), jnp.float32))
        out_specs.append(row_spec)
        out_shapes.append(jax.ShapeDtypeStruct((B, 1), jnp.float32))
        out_specs.append(col_spec)

    outs = pl.pallas_call(
        functools.partial(_scn_kernel, depth, TB, V),
        grid=grid,
        in_specs=[
            row_spec,
            fix_spec((1, V)),
            fix_spec((1, V)),
            fix_spec((depth, V)),
            pl.BlockSpec(memory_space=pltpu.SMEM),
        ],
        out_specs=out_specs,
        out_shape=out_shapes,
        compiler_params=pltpu.CompilerParams(
            dimension_semantics=("arbitrary",),
        ),
    )(inp_p, vu, vf, L, biases)
    return outs


def kernel(inp, visible_units, visible_fs, biases, L):
    B = inp.shape[0]
    V = visible_units.shape[0]
    depth = L.shape[0]
    TB = 2048
    while B % TB:
        TB //= 2
    outs = _scn(inp, visible_units, visible_fs, biases, L, TB)
    out = outs[0].reshape(B, 1, 1)
    res = [out]
    for i in range(depth):
        res.append(outs[1 + 2 * i].reshape(B, V, 1))
        res.append(outs[2 + 2 * i])
    return tuple(res)


# speedup vs baseline: 11.4238x; 1.0562x over previous
import functools

import jax
import jax.numpy as jnp
from jax.experimental import pallas as pl
from jax.experimental.pallas import tpu as pltpu


def _scn_kernel(depth, TB, V, inp_ref, vu_ref, vf_ref, L_ref, LT_ref, b_ref, *out_refs):
    out_ref = out_refs[0]
    h_refs = out_refs[1::2]
    n_refs = out_refs[2::2]

    w_p = inp_ref[...]
    s = jnp.sum(w_p, axis=1, keepdims=True)
    iota = jax.lax.broadcasted_iota(jnp.int32, (TB, V), 1).astype(jnp.float32)
    w = jnp.where(iota == 0.0, 1.0 - s, w_p)
    f = jnp.broadcast_to(vf_ref[...], (TB, V))
    h = jnp.broadcast_to(vu_ref[...], (TB, V))

    for i in range(depth):
        l = L_ref[i : i + 1, :]
        wd = w / (l + 1e-20)
        m = jnp.min(wd, axis=1, keepdims=True)
        cand = jnp.where(wd == m, iota, float(V))
        idx = jnp.min(cand, axis=1, keepdims=True)
        mask = iota == idx
        lb = LT_ref[:, i : i + 1].astype(jnp.bfloat16)
        s_h = jnp.dot(h.astype(jnp.bfloat16), lb, preferred_element_type=jnp.float32)
        s_f = jnp.dot(f.astype(jnp.bfloat16), lb, preferred_element_type=jnp.float32) + b_ref[i, 0]
        h_refs[i][...] = h
        n_refs[i][...] = s_h
        w = jnp.where(mask, m, w - m * l)
        h = jnp.where(mask, s_h, h)
        f = jnp.where(mask, s_f, f)

    out_ref[...] = jnp.sum(w * f, axis=1, keepdims=True)


@functools.partial(jax.jit, static_argnames=("TB",))
def _scn(inp, visible_units, visible_fs, biases, L, TB):
    B = inp.shape[0]
    V = visible_units.shape[0]
    depth = L.shape[0]
    inp_p = jnp.pad(inp, ((0, 0), (1, 0)))
    vu = visible_units.reshape(1, V)
    vf = visible_fs.reshape(1, V)
    LT = L.T

    grid = (B // TB,)
    row_spec = pl.BlockSpec((TB, V), lambda b: (b, 0))
    col_spec = pl.BlockSpec((TB, 1), lambda b: (b, 0))
    fix_spec = lambda shape: pl.BlockSpec(shape, lambda b: (0, 0))

    out_shapes = [jax.ShapeDtypeStruct((B, 1), jnp.float32)]
    out_specs = [col_spec]
    for _ in range(depth):
        out_shapes.append(jax.ShapeDtypeStruct((B, V), jnp.float32))
        out_specs.append(row_spec)
        out_shapes.append(jax.ShapeDtypeStruct((B, 1), jnp.float32))
        out_specs.append(col_spec)

    outs = pl.pallas_call(
        functools.partial(_scn_kernel, depth, TB, V),
        grid=grid,
        in_specs=[
            row_spec,
            fix_spec((1, V)),
            fix_spec((1, V)),
            fix_spec((depth, V)),
            fix_spec((V, depth)),
            pl.BlockSpec(memory_space=pltpu.SMEM),
        ],
        out_specs=out_specs,
        out_shape=out_shapes,
        compiler_params=pltpu.CompilerParams(
            dimension_semantics=("arbitrary",),
        ),
    )(inp_p, vu, vf, L, LT, biases)
    return outs


def kernel(inp, visible_units, visible_fs, biases, L):
    B = inp.shape[0]
    V = visible_units.shape[0]
    depth = L.shape[0]
    TB = 2048
    while B % TB:
        TB //= 2
    outs = _scn(inp, visible_units, visible_fs, biases, L, TB)
    out = outs[0].reshape(B, 1, 1)
    res = [out]
    for i in range(depth):
        res.append(outs[1 + 2 * i].reshape(B, V, 1))
        res.append(outs[2 + 2 * i])
    return tuple(res)


# speedup vs baseline: 32.6002x; 2.8537x over previous
import functools

import jax
import jax.numpy as jnp
from jax.experimental import pallas as pl
from jax.experimental.pallas import tpu as pltpu


def _scn_kernel(depth, TB, V, inp_ref, vu_ref, vf_ref, L_ref, LT_ref, b_ref, *out_refs):
    out_ref = out_refs[0]
    h_refs = out_refs[1::2]
    n_refs = out_refs[2::2]

    it = inp_ref[...]
    s = jnp.sum(it, axis=0, keepdims=True)
    w = jnp.concatenate([1.0 - s, it], axis=0)
    iota = jax.lax.broadcasted_iota(jnp.int32, (V, TB), 0).astype(jnp.float32)
    f = jnp.broadcast_to(vf_ref[...], (V, TB))
    h = jnp.broadcast_to(vu_ref[...], (V, TB))

    for i in range(depth):
        lc = LT_ref[:, i : i + 1]
        wd = w / (lc + 1e-20)
        m = jnp.min(wd, axis=0, keepdims=True)
        cand = jnp.where(wd == m, iota, float(V))
        idx = jnp.min(cand, axis=0, keepdims=True)
        mask = iota == idx
        lb = L_ref[i : i + 1, :].astype(jnp.bfloat16)
        s_h = jnp.dot(lb, h.astype(jnp.bfloat16), preferred_element_type=jnp.float32)
        s_f = jnp.dot(lb, f.astype(jnp.bfloat16), preferred_element_type=jnp.float32) + b_ref[i, 0]
        h_refs[i][...] = h
        n_refs[i][...] = s_h
        w = jnp.where(mask, m, w - m * lc)
        h = jnp.where(mask, s_h, h)
        f = jnp.where(mask, s_f, f)

    out_ref[...] = jnp.sum(w * f, axis=0, keepdims=True)


@functools.partial(jax.jit, static_argnames=("TB",))
def _scn(inp, visible_units, visible_fs, biases, L, TB):
    B = inp.shape[0]
    V = visible_units.shape[0]
    depth = L.shape[0]
    inp_t = inp.T
    vu = visible_units
    vf = visible_fs
    LT = L.T

    grid = (B // TB,)
    row_spec = pl.BlockSpec((V, TB), lambda b: (0, b))
    col_spec = pl.BlockSpec((1, TB), lambda b: (0, b))
    fix_spec = lambda shape: pl.BlockSpec(shape, lambda b: (0, 0))

    out_shapes = [jax.ShapeDtypeStruct((1, B), jnp.float32)]
    out_specs = [col_spec]
    for _ in range(depth):
        out_shapes.append(jax.ShapeDtypeStruct((V, B), jnp.float32))
        out_specs.append(row_spec)
        out_shapes.append(jax.ShapeDtypeStruct((1, B), jnp.float32))
        out_specs.append(col_spec)

    outs = pl.pallas_call(
        functools.partial(_scn_kernel, depth, TB, V),
        grid=grid,
        in_specs=[
            pl.BlockSpec((V - 1, TB), lambda b: (0, b)),
            fix_spec((V, 1)),
            fix_spec((V, 1)),
            fix_spec((depth, V)),
            fix_spec((V, depth)),
            pl.BlockSpec(memory_space=pltpu.SMEM),
        ],
        out_specs=out_specs,
        out_shape=out_shapes,
        compiler_params=pltpu.CompilerParams(
            dimension_semantics=("arbitrary",),
        ),
    )(inp_t, vu, vf, L, LT, biases)
    return outs


def kernel(inp, visible_units, visible_fs, biases, L):
    B = inp.shape[0]
    V = visible_units.shape[0]
    depth = L.shape[0]
    TB = 2048
    while B % TB:
        TB //= 2
    outs = _scn(inp, visible_units, visible_fs, biases, L, TB)
    out = outs[0].T.reshape(B, 1, 1)
    res = [out]
    for i in range(depth):
        res.append(outs[1 + 2 * i].T.reshape(B, V, 1))
        res.append(outs[2 + 2 * i].T)
    return tuple(res)


# speedup vs baseline: 61.9169x; 1.8993x over previous
import functools

import jax
import jax.numpy as jnp
from jax.experimental import pallas as pl
from jax.experimental.pallas import tpu as pltpu


def _scn_kernel(depth, TB, V, inp_ref, vu_ref, vf_ref, L_ref, LT_ref, b_ref, *out_refs):
    out_ref = out_refs[0]
    h_refs = out_refs[1::2]
    n_refs = out_refs[2::2]

    it = inp_ref[...]
    s = jnp.sum(it, axis=0, keepdims=True)
    w = jnp.concatenate([1.0 - s, it], axis=0)
    iota = jax.lax.broadcasted_iota(jnp.int32, (V, TB), 0).astype(jnp.float32)
    f = jnp.broadcast_to(vf_ref[...], (V, TB))
    h = jnp.broadcast_to(vu_ref[...], (V, TB))

    for i in range(depth):
        lc = LT_ref[:, i : i + 1]
        wd = w / (lc + 1e-20)
        m = jnp.min(wd, axis=0, keepdims=True)
        cand = jnp.where(wd == m, iota, float(V))
        idx = jnp.min(cand, axis=0, keepdims=True)
        mask = iota == idx
        lb = L_ref[i : i + 1, :].astype(jnp.bfloat16)
        s_h = jnp.dot(lb, h.astype(jnp.bfloat16), preferred_element_type=jnp.float32)
        s_f = jnp.dot(lb, f.astype(jnp.bfloat16), preferred_element_type=jnp.float32) + b_ref[i, 0]
        h_refs[i][...] = h.reshape(V, TB // 128, 128)
        n_refs[i][...] = s_h
        w = jnp.where(mask, m, w - m * lc)
        h = jnp.where(mask, s_h, h)
        f = jnp.where(mask, s_f, f)

    out_ref[...] = jnp.sum(w * f, axis=0, keepdims=True)


@functools.partial(jax.jit, static_argnames=("TB",))
def _scn(inp, visible_units, visible_fs, biases, L, TB):
    B = inp.shape[0]
    V = visible_units.shape[0]
    depth = L.shape[0]
    inp_t = inp.T
    vu = visible_units
    vf = visible_fs
    LT = L.T

    grid = (B // TB,)
    row_spec = pl.BlockSpec((V, TB // 128, 128), lambda b: (0, b, 0))
    col_spec = pl.BlockSpec((1, TB), lambda b: (0, b))
    fix_spec = lambda shape: pl.BlockSpec(shape, lambda b: (0, 0))

    out_shapes = [jax.ShapeDtypeStruct((1, B), jnp.float32)]
    out_specs = [col_spec]
    for _ in range(depth):
        out_shapes.append(jax.ShapeDtypeStruct((V, B // 128, 128), jnp.float32))
        out_specs.append(row_spec)
        out_shapes.append(jax.ShapeDtypeStruct((1, B), jnp.float32))
        out_specs.append(col_spec)

    outs = pl.pallas_call(
        functools.partial(_scn_kernel, depth, TB, V),
        grid=grid,
        in_specs=[
            pl.BlockSpec((V - 1, TB), lambda b: (0, b)),
            fix_spec((V, 1)),
            fix_spec((V, 1)),
            fix_spec((depth, V)),
            fix_spec((V, depth)),
            pl.BlockSpec(memory_space=pltpu.SMEM),
        ],
        out_specs=out_specs,
        out_shape=out_shapes,
        compiler_params=pltpu.CompilerParams(
            dimension_semantics=("arbitrary",),
        ),
    )(inp_t, vu, vf, L, LT, biases)
    return outs


def kernel(inp, visible_units, visible_fs, biases, L):
    B = inp.shape[0]
    V = visible_units.shape[0]
    depth = L.shape[0]
    TB = 2048
    while B % TB:
        TB //= 2
    outs = _scn(inp, visible_units, visible_fs, biases, L, TB)
    out = outs[0].T.reshape(B, 1, 1)
    res = [out]
    for i in range(depth):
        res.append(outs[1 + 2 * i].transpose(1, 2, 0).reshape(B, V, 1))
        res.append(outs[2 + 2 * i].T)
    return tuple(res)


# speedup vs baseline: 62.0455x; 1.0021x over previous
import functools

import jax
import jax.numpy as jnp
from jax.experimental import pallas as pl
from jax.experimental.pallas import tpu as pltpu


def _scn_kernel(depth, TB, V, inp_ref, vu_ref, vf_ref, L_ref, LT_ref, b_ref, *out_refs):
    out_ref = out_refs[0]
    h_refs = out_refs[1::2]
    n_refs = out_refs[2::2]

    it = inp_ref[...]
    s = jnp.sum(it, axis=0, keepdims=True)
    w = jnp.concatenate([1.0 - s, it], axis=0)
    iota = jax.lax.broadcasted_iota(jnp.int32, (V, TB), 0).astype(jnp.float32)
    f = jnp.broadcast_to(vf_ref[...], (V, TB))
    h = jnp.broadcast_to(vu_ref[...], (V, TB))

    for i in range(depth):
        lc = LT_ref[:, i : i + 1]
        wd = w / (lc + 1e-20)
        m = jnp.min(wd, axis=0, keepdims=True)
        cand = jnp.where(wd == m, iota, float(V))
        idx = jnp.min(cand, axis=0, keepdims=True)
        mask = iota == idx
        lb = L_ref[i : i + 1, :].astype(jnp.bfloat16)
        s_h = jnp.dot(lb, h.astype(jnp.bfloat16), preferred_element_type=jnp.float32)
        s_f = jnp.dot(lb, f.astype(jnp.bfloat16), preferred_element_type=jnp.float32) + b_ref[i, 0]
        h_refs[i][...] = h.reshape(V, TB // 128, 128)
        n_refs[i][...] = s_h
        w = jnp.where(mask, m, w - m * lc)
        h = jnp.where(mask, s_h, h)
        f = jnp.where(mask, s_f, f)

    out_ref[...] = jnp.sum(w * f, axis=0, keepdims=True)


@functools.partial(jax.jit, static_argnames=("TB",))
def _scn(inp, visible_units, visible_fs, biases, L, TB):
    B = inp.shape[0]
    V = visible_units.shape[0]
    depth = L.shape[0]
    inp_t = inp.T
    vu = visible_units
    vf = visible_fs
    LT = L.T

    grid = (B // TB,)
    row_spec = pl.BlockSpec((V, TB // 128, 128), lambda b: (0, b, 0))
    col_spec = pl.BlockSpec((1, TB), lambda b: (0, b))
    fix_spec = lambda shape: pl.BlockSpec(shape, lambda b: (0, 0))

    out_shapes = [jax.ShapeDtypeStruct((1, B), jnp.float32)]
    out_specs = [col_spec]
    for _ in range(depth):
        out_shapes.append(jax.ShapeDtypeStruct((V, B // 128, 128), jnp.float32))
        out_specs.append(row_spec)
        out_shapes.append(jax.ShapeDtypeStruct((1, B), jnp.float32))
        out_specs.append(col_spec)

    outs = pl.pallas_call(
        functools.partial(_scn_kernel, depth, TB, V),
        grid=grid,
        in_specs=[
            pl.BlockSpec((V - 1, TB), lambda b: (0, b)),
            fix_spec((V, 1)),
            fix_spec((V, 1)),
            fix_spec((depth, V)),
            fix_spec((V, depth)),
            pl.BlockSpec(memory_space=pltpu.SMEM),
        ],
        out_specs=out_specs,
        out_shape=out_shapes,
        compiler_params=pltpu.CompilerParams(
            dimension_semantics=("parallel",),
        ),
    )(inp_t, vu, vf, L, LT, biases)
    return outs


def kernel(inp, visible_units, visible_fs, biases, L):
    B = inp.shape[0]
    V = visible_units.shape[0]
    depth = L.shape[0]
    TB = 2048
    while B % TB:
        TB //= 2
    outs = _scn(inp, visible_units, visible_fs, biases, L, TB)
    out = outs[0].T.reshape(B, 1, 1)
    res = [out]
    for i in range(depth):
        res.append(outs[1 + 2 * i].transpose(1, 2, 0).reshape(B, V, 1))
        res.append(outs[2 + 2 * i].T)
    return tuple(res)


# speedup vs baseline: 66.4353x; 1.0708x over previous
import functools

import jax
import jax.numpy as jnp
from jax.experimental import pallas as pl
from jax.experimental.pallas import tpu as pltpu


def _scn_kernel(depth, TB, V, inp_ref, vu_ref, vf_ref, vu3_ref, L_ref, LT_ref, b_ref, *out_refs):
    out_ref = out_refs[0]
    h_refs = out_refs[1::2]
    n_refs = out_refs[2::2]
    RB = TB // 128

    it = inp_ref[...]
    s = jnp.sum(it, axis=0, keepdims=True)
    w = jnp.concatenate([1.0 - s, it], axis=0)
    iota = jax.lax.broadcasted_iota(jnp.int32, (V, TB), 0).astype(jnp.float32)
    f = jnp.broadcast_to(vf_ref[...], (V, TB))
    h = jnp.broadcast_to(vu_ref[...], (V, TB))
    h3 = jnp.broadcast_to(vu3_ref[...], (V, RB, 128))
    iota3 = jax.lax.broadcasted_iota(jnp.int32, (V, RB, 128), 0)

    for i in range(depth):
        lc = LT_ref[:, i : i + 1]
        wd = w / (lc + 1e-20)
        m = jnp.min(wd, axis=0, keepdims=True)
        cand = jnp.where(wd == m, iota, float(V))
        idx = jnp.min(cand, axis=0, keepdims=True)
        mask = iota == idx
        lb = L_ref[i : i + 1, :].astype(jnp.bfloat16)
        s_h = jnp.dot(lb, h.astype(jnp.bfloat16), preferred_element_type=jnp.float32)
        s_f = jnp.dot(lb, f.astype(jnp.bfloat16), preferred_element_type=jnp.float32) + b_ref[i, 0]
        h_refs[i][...] = h3
        n_refs[i][...] = s_h
        w = jnp.where(mask, m, w - m * lc)
        h = jnp.where(mask, s_h, h)
        f = jnp.where(mask, s_f, f)
        h3 = jnp.where(
            iota3 == idx.reshape(1, RB, 128).astype(jnp.int32),
            s_h.reshape(1, RB, 128),
            h3,
        )

    out_ref[...] = jnp.sum(w * f, axis=0, keepdims=True)


@functools.partial(jax.jit, static_argnames=("TB",))
def _scn(inp, visible_units, visible_fs, biases, L, TB):
    B = inp.shape[0]
    V = visible_units.shape[0]
    depth = L.shape[0]
    inp_t = inp.T
    vu = visible_units
    vf = visible_fs
    vu3 = jnp.broadcast_to(visible_units[:, :, None], (V, 1, 128))
    LT = L.T

    grid = (B // TB,)
    row_spec = pl.BlockSpec((V, TB // 128, 128), lambda b: (0, b, 0))
    col_spec = pl.BlockSpec((1, TB), lambda b: (0, b))
    fix_spec = lambda shape: pl.BlockSpec(shape, lambda b: (0, 0))

    out_shapes = [jax.ShapeDtypeStruct((1, B), jnp.float32)]
    out_specs = [col_spec]
    for _ in range(depth):
        out_shapes.append(jax.ShapeDtypeStruct((V, B // 128, 128), jnp.float32))
        out_specs.append(row_spec)
        out_shapes.append(jax.ShapeDtypeStruct((1, B), jnp.float32))
        out_specs.append(col_spec)

    outs = pl.pallas_call(
        functools.partial(_scn_kernel, depth, TB, V),
        grid=grid,
        in_specs=[
            pl.BlockSpec((V - 1, TB), lambda b: (0, b)),
            fix_spec((V, 1)),
            fix_spec((V, 1)),
            pl.BlockSpec((V, 1, 128), lambda b: (0, 0, 0)),
            fix_spec((depth, V)),
            fix_spec((V, depth)),
            pl.BlockSpec(memory_space=pltpu.SMEM),
        ],
        out_specs=out_specs,
        out_shape=out_shapes,
        compiler_params=pltpu.CompilerParams(
            dimension_semantics=("parallel",),
        ),
    )(inp_t, vu, vf, vu3, L, LT, biases)
    return outs


def kernel(inp, visible_units, visible_fs, biases, L):
    B = inp.shape[0]
    V = visible_units.shape[0]
    depth = L.shape[0]
    TB = 2048
    while B % TB:
        TB //= 2
    outs = _scn(inp, visible_units, visible_fs, biases, L, TB)
    out = outs[0].T.reshape(B, 1, 1)
    res = [out]
    for i in range(depth):
        res.append(outs[1 + 2 * i].transpose(1, 2, 0).reshape(B, V, 1))
        res.append(outs[2 + 2 * i].T)
    return tuple(res)
